# Initial kernel scaffold; baseline (speedup 1.0000x reference)
import jax
import jax.numpy as jnp
from jax.experimental import pallas as pl


def kernel(feat0, feat1, e_feat, edge_index, Wfc0, bfc0, Wfc1, bfc1, W0, al0, ar0, b0, W1, al1, ar1, b1, W2, al2, ar2, b2, resW2):
    raise NotImplementedError("write your pallas kernel here")


# speedup vs baseline: 24.2089x; 24.2089x over previous
import functools
import jax
import jax.numpy as jnp
from jax import lax
from jax.experimental import pallas as pl
from jax.experimental.pallas import tpu as pltpu
from jax.experimental.pallas import tpu_sc as plsc

N = 10000
NP = 10112
E = 320000
D = 128
NCLS = 16
NEG = 0.2
NTILES = 32
EPT = E // NTILES
EPTP = 10112
NCHUNK = EPTP // 16
PAD_IDX = 10016
RB = NP // 16
RING = 8

_f32 = jnp.float32
_i32 = jnp.int32


def _dot(a, b):
  return jnp.dot(a, b, preferred_element_type=_f32)


def _attn_tail(f, al_ref, ar_ref, el_ref, er_ref, mv_ref, mel_s, mer_s, i):
  el = jnp.sum(f * al_ref[...], axis=1)
  er = jnp.sum(f * ar_ref[...], axis=1)
  el_ref[...] = el[:, None]
  er_ref[...] = er[:, None]

  @pl.when(i == 0)
  def _():
    mel_s[0, 0] = -jnp.inf
    mer_s[0, 0] = -jnp.inf

  mel = jnp.maximum(mel_s[0, 0], jnp.max(el))
  mer = jnp.maximum(mer_s[0, 0], jnp.max(er))
  mel_s[0, 0] = mel
  mer_s[0, 0] = mer
  mv_ref[...] = jnp.full((1, 128), mel + mer, _f32)


def _tc0_body(f0_ref, f1_ref, Wf0_ref, bf0_ref, Wf1_ref, bf1_ref, W_ref,
              al_ref, ar_ref, f_ref, el_ref, er_ref, mv_ref, mel_s, mer_s):
  i = pl.program_id(0)
  use0 = i < 6
  xb = jnp.where(use0, f0_ref[...], f1_ref[...])
  Wf = jnp.where(use0, Wf0_ref[...], Wf1_ref[...])
  bf = jnp.where(use0, bf0_ref[...], bf1_ref[...])
  h = _dot(xb, Wf) + bf
  f = _dot(h, W_ref[...])
  f_ref[...] = f
  _attn_tail(f, al_ref, ar_ref, el_ref, er_ref, mv_ref, mel_s, mer_s, i)


def _tc0(feat0, feat1, Wfc0, bfc0, Wfc1, bfc1, W0, al0, ar0):
  full = lambda r, c: pl.BlockSpec((r, c), lambda i: (0, 0))
  return pl.pallas_call(
      _tc0_body,
      grid=(10,),
      in_specs=[
          pl.BlockSpec((1000, D), lambda i: (jnp.minimum(i, 5), 0)),
          pl.BlockSpec((1000, D), lambda i: (jnp.maximum(i - 6, 0), 0)),
          full(D, D), full(1, D), full(D, D), full(1, D), full(D, D),
          full(1, D), full(1, D),
      ],
      out_specs=[
          pl.BlockSpec((1000, D), lambda i: (i, 0)),
          pl.BlockSpec((1000, 1), lambda i: (i, 0)),
          pl.BlockSpec((1000, 1), lambda i: (i, 0)),
          pl.BlockSpec((1, 128), lambda i: (0, 0)),
      ],
      out_shape=[
          jax.ShapeDtypeStruct((NP, D), _f32),
          jax.ShapeDtypeStruct((NP, 1), _f32),
          jax.ShapeDtypeStruct((NP, 1), _f32),
          jax.ShapeDtypeStruct((1, 128), _f32),
      ],
      scratch_shapes=[pltpu.SMEM((1, 1), _f32), pltpu.SMEM((1, 1), _f32)],
  )(feat0, feat1, Wfc0, bfc0, Wfc1, bfc1, W0, al0, ar0)


def _tc_mid(Do, with_res, with_resw):
  def body(*refs):
    it = iter(refs)
    A_ref = next(it)
    dA_ref = next(it)
    dB_ref = next(it)
    res_ref = next(it) if with_res else None
    b_ref = next(it)
    W_ref = next(it)
    al_ref = next(it)
    ar_ref = next(it)
    resW_ref = next(it) if with_resw else None
    h_ref = next(it)
    f_ref = next(it)
    res2_ref = next(it) if with_resw else None
    el_ref = next(it)
    er_ref = next(it)
    mv_ref = next(it)
    mel_s = next(it)
    mer_s = next(it)
    i = pl.program_id(0)

    den = dA_ref[:, 0] + dB_ref[:, 0]
    den = jnp.where(den > 0.0, den, 1.0)
    rst = (A_ref[0] + A_ref[1]) / den[:, None] + b_ref[...]
    if with_res:
      rst = rst + res_ref[...]
    h = jnp.where(rst > 0.0, rst, jnp.exp(jnp.minimum(rst, 0.0)) - 1.0)
    f = _dot(h, W_ref[...])
    h_ref[...] = h
    if Do < 128:
      f_ref[...] = jnp.concatenate(
          [f, jnp.zeros((f.shape[0], 128 - Do), _f32)], axis=1)
    else:
      f_ref[...] = f
    if with_resw:
      res2_ref[...] = _dot(h, resW_ref[...])
    _attn_tail(f, al_ref, ar_ref, el_ref, er_ref, mv_ref, mel_s, mer_s, i)

  def run(A, dA, dB, res, b, W, al, ar, resW):
    full = lambda r, c: pl.BlockSpec((r, c), lambda i: (0, 0))
    in_specs = [
        pl.BlockSpec((2, 1000, D), lambda i: (0, i, 0)),
        pl.BlockSpec((1000, 1), lambda i: (i, 0)),
        pl.BlockSpec((1000, 1), lambda i: (i, 0)),
    ]
    args = [A, dA, dB]
    if with_res:
      in_specs.append(pl.BlockSpec((1000, D), lambda i: (i, 0)))
      args.append(res)
    in_specs += [full(1, D), full(D, Do), full(1, Do), full(1, Do)]
    args += [b, W, al, ar]
    if with_resw:
      in_specs.append(full(D, Do))
      args.append(resW)
    out_specs = [
        pl.BlockSpec((1000, D), lambda i: (i, 0)),
        pl.BlockSpec((1000, 128), lambda i: (i, 0)),
    ]
    out_shape = [
        jax.ShapeDtypeStruct((NP, D), _f32),
        jax.ShapeDtypeStruct((NP, 128), _f32),
    ]
    if with_resw:
      out_specs.append(pl.BlockSpec((1000, Do), lambda i: (i, 0)))
      out_shape.append(jax.ShapeDtypeStruct((NP, Do), _f32))
    out_specs += [
        pl.BlockSpec((1000, 1), lambda i: (i, 0)),
        pl.BlockSpec((1000, 1), lambda i: (i, 0)),
        pl.BlockSpec((1, 128), lambda i: (0, 0)),
    ]
    out_shape += [
        jax.ShapeDtypeStruct((NP, 1), _f32),
        jax.ShapeDtypeStruct((NP, 1), _f32),
        jax.ShapeDtypeStruct((1, 128), _f32),
    ]
    return pl.pallas_call(
        body,
        grid=(10,),
        in_specs=in_specs,
        out_specs=out_specs,
        out_shape=out_shape,
        scratch_shapes=[pltpu.SMEM((1, 1), _f32), pltpu.SMEM((1, 1), _f32)],
    )(*args)

  return run


def _tc_final_body(A_ref, dA_ref, dB_ref, res_ref, b_ref, out_ref):
  den = dA_ref[:, 0] + dB_ref[:, 0]
  den = jnp.where(den > 0.0, den, 1.0)
  out_ref[...] = (A_ref[0] + A_ref[1]) / den[:, None] + res_ref[...] + b_ref[...]


def _tc_final(A, dA, dB, res, b):
  return pl.pallas_call(
      _tc_final_body,
      grid=(10,),
      in_specs=[
          pl.BlockSpec((2, 1000, NCLS), lambda i: (0, i, 0)),
          pl.BlockSpec((1000, 1), lambda i: (i, 0)),
          pl.BlockSpec((1000, 1), lambda i: (i, 0)),
          pl.BlockSpec((1000, NCLS), lambda i: (i, 0)),
          pl.BlockSpec((1, NCLS), lambda i: (0, 0)),
      ],
      out_specs=pl.BlockSpec((1000, NCLS), lambda i: (i, 0)),
      out_shape=jax.ShapeDtypeStruct((NP, NCLS), _f32),
  )(A, dA, dB, res, b)


_mesh = plsc.VectorSubcoreMesh(core_axis_name="c", subcore_axis_name="s")
_sc_params = pltpu.CompilerParams(needs_layout_passes=False)


def _sc_pass1():
  out_type = [
      jax.ShapeDtypeStruct((NTILES * EPTP,), _f32),
      jax.ShapeDtypeStruct((NP,), _f32),
      jax.ShapeDtypeStruct((NP,), _f32),
  ]
  scratch = (
      [
          pltpu.VMEM((NP,), _f32),
          pltpu.VMEM((NP,), _f32),
          pltpu.VMEM((16,), _f32),
          pltpu.VMEM((EPTP,), _i32),
          pltpu.VMEM((EPTP,), _i32),
          pltpu.VMEM((EPTP,), _f32),
          pltpu.VMEM((RB + 8,), _f32),
          pltpu.VMEM_SHARED((NP,), _f32),
      ]
      + [pltpu.SemaphoreType.DMA for _ in range(RING)]
  )

  @functools.partial(pl.kernel, mesh=_mesh, out_type=out_type,
                     scratch_types=scratch, compiler_params=_sc_params)
  def pass1(el_h, er_h, mv_h, src_h, dst_h, ee_all_h, denA_h, denB_h, *sc):
    el_v, er_v, mv_v, src_v, dst_v, ee_v, zb_v, den_sh = sc[:8]
    dsem = sc[8:8 + RING]

    c = lax.axis_index("c")
    s = lax.axis_index("s")
    wid = c * 16 + s

    pltpu.sync_copy(el_h, el_v)
    pltpu.sync_copy(er_h, er_v)
    pltpu.sync_copy(mv_h, mv_v)
    pltpu.sync_copy(src_h.at[pl.ds(wid * EPTP, EPTP)], src_v)
    pltpu.sync_copy(dst_h.at[pl.ds(wid * EPTP, EPTP)], dst_v)

    zero16 = jnp.zeros((16,), _f32)
    for q in range((RB + 8) // 16):
      zb_v[pl.ds(q * 16, 16)] = zero16
    obase = s * RB
    pltpu.sync_copy(zb_v.at[pl.ds(0, RB)], den_sh.at[pl.ds(obase, RB)])
    plsc.subcore_barrier()

    mv = mv_v[...]

    def dwait(b):
      dstv0 = dst_v[pl.ds(0, 16)]
      pltpu.make_async_copy(ee_v.at[pl.ds(0, 16)], den_sh.at[dstv0],
                            dsem[b]).wait()

    def visit(chunk, b):
      srcv = src_v[pl.ds(chunk * 16, 16)]
      dstv = dst_v[pl.ds(chunk * 16, 16)]
      el_s = plsc.load_gather(el_v, [srcv])
      er_d = plsc.load_gather(er_v, [dstv])
      x = el_s + er_d
      lk = jnp.where(x > 0.0, x, NEG * x)
      ee = jnp.exp(lk - mv)
      ee_v[pl.ds(chunk * 16, 16)] = ee
      pltpu.async_copy(ee_v.at[pl.ds(chunk * 16, 16)], den_sh.at[dstv],
                       dsem[b], add=True)

    for k in range(RING):
      visit(k, k)

    def main_body(t, _):
      cbase = t * RING
      for k in range(RING):
        dwait(k)
        visit(cbase + k, k)
      return 0

    lax.fori_loop(1, NCHUNK // RING, main_body, 0)

    for k in range(RING):
      dwait(k)

    pltpu.sync_copy(ee_v, ee_all_h.at[pl.ds(wid * EPTP, EPTP)])
    plsc.subcore_barrier()

    @pl.when(jnp.logical_and(c == 0, s == 0))
    def _():
      pltpu.sync_copy(den_sh, denA_h)

    @pl.when(jnp.logical_and(c == 1, s == 0))
    def _():
      pltpu.sync_copy(den_sh, denB_h)

  return pass1


def _sc_pass2(Dd):
  out_type = jax.ShapeDtypeStruct((2, NP, Dd), _f32)
  scratch = (
      [
          pltpu.VMEM((EPTP,), _i32),
          pltpu.VMEM((EPTP,), _i32),
          pltpu.VMEM((EPTP + 16,), _f32),
          pltpu.VMEM((8, Dd), _f32),
      ]
      + [pltpu.VMEM((16, Dd), _f32) for _ in range(RING)]
      + [pltpu.VMEM_SHARED((NP, Dd), _f32)]
      + [pltpu.SemaphoreType.DMA for _ in range(2 * RING)]
  )

  @functools.partial(pl.kernel, mesh=_mesh, out_type=out_type,
                     scratch_types=scratch, compiler_params=_sc_params)
  def pass2(src_h, dst_h, ee_all_h, feat_h, rst_h, *sc):
    src_v, dst_v, ee_v, zrow_v = sc[:4]
    rows = sc[4:4 + RING]
    acc_sh = sc[4 + RING]
    gsem = sc[5 + RING:5 + 2 * RING]
    ssem = sc[5 + 2 * RING:5 + 3 * RING]

    c = lax.axis_index("c")
    s = lax.axis_index("s")
    wid = c * 16 + s

    pltpu.sync_copy(src_h.at[pl.ds(wid * EPTP, EPTP)], src_v)
    pltpu.sync_copy(dst_h.at[pl.ds(wid * EPTP, EPTP)], dst_v)
    pltpu.sync_copy(ee_all_h.at[pl.ds(wid * EPTP, EPTP)],
                    ee_v.at[pl.ds(16, EPTP)])

    zero16 = jnp.zeros((16,), _f32)
    for r in range(8):
      for q in range(Dd // 16):
        zrow_v[r, pl.ds(q * 16, 16)] = zero16
    obase = s * RB
    for j in range(RB // 8):
      pltpu.sync_copy(zrow_v, acc_sh.at[pl.ds(obase + j * 8, 8)])
    plsc.subcore_barrier()

    def start_gather(chunk, b):
      srcv = src_v[pl.ds(chunk * 16, 16)]
      pltpu.make_async_copy(feat_h.at[srcv], rows[b], gsem[b]).start()

    def wait_gather(b):
      srcv = src_v[pl.ds(0, 16)]
      pltpu.make_async_copy(feat_h.at[srcv], rows[b], gsem[b]).wait()

    def wait_scatter(b):
      dstv = dst_v[pl.ds(0, 16)]
      pltpu.make_async_copy(rows[b], acc_sh.at[dstv], ssem[b]).wait()

    def visit(chunk, b1, do_scwait, do_gissue):
      b2 = (b1 + 4) % RING
      wait_gather(b1)
      dstv = dst_v[pl.ds(chunk * 16, 16)]
      ebase = chunk * 16 + 16
      for r in range(16):
        w = plsc.load_gather(ee_v, [jnp.full((16,), r, _i32) + ebase])
        for q in range(Dd // 16):
          sl = pl.ds(q * 16, 16)
          rows[b1][r, sl] = rows[b1][r, sl] * w
      pltpu.async_copy(rows[b1], acc_sh.at[dstv], ssem[b1], add=True)
      if do_scwait:
        wait_scatter(b2)
      if do_gissue:
        start_gather(chunk + 4, b2)

    for cc in range(4):
      start_gather(cc, cc)
    for cc in range(4):
      visit(cc, cc, do_scwait=False, do_gissue=True)

    def main_body(t, _):
      cbase = 4 + t * 8
      for k in range(8):
        visit(cbase + k, (4 + k) % RING, do_scwait=True, do_gissue=True)
      return 0

    lax.fori_loop(0, (NCHUNK - 8) // 8, main_body, 0)

    for k in range(4):
      visit(NCHUNK - 4 + k, (4 + k) % RING, do_scwait=True, do_gissue=False)
    for b in range(4, 8):
      wait_scatter(b)

    plsc.subcore_barrier()

    pltpu.sync_copy(acc_sh.at[pl.ds(obase, RB)],
                    rst_h.at[c, pl.ds(obase, RB)])

  return pass2


_scp1 = _sc_pass1()
_scp2_128 = _sc_pass2(D)
_tc1 = _tc_mid(D, with_res=False, with_resw=False)
_tc2 = _tc_mid(NCLS, with_res=True, with_resw=True)


def kernel(feat0, feat1, e_feat, edge_index, Wfc0, bfc0, Wfc1, bfc1,
           W0, al0, ar0, b0, W1, al1, ar1, b1, W2, al2, ar2, b2, resW2):
  src = edge_index[0]
  dst = edge_index[1]
  padcol = jnp.full((NTILES, EPTP - EPT), PAD_IDX, _i32)
  srcp = jnp.concatenate([src.reshape(NTILES, EPT), padcol], axis=1).reshape(-1)
  dstp = jnp.concatenate([dst.reshape(NTILES, EPT), padcol], axis=1).reshape(-1)

  f0t, el0, er0, mv0 = _tc0(feat0, feat1, Wfc0, bfc0.reshape(1, D),
                            Wfc1, bfc1.reshape(1, D), W0, al0, ar0)
  ee0, dA0, dB0 = _scp1(el0.reshape(NP), er0.reshape(NP), mv0[0, :16],
                        srcp, dstp)
  rst0 = _scp2_128(srcp, dstp, ee0, f0t)
  h1, f1t, el1, er1, mv1 = _tc1(rst0, dA0.reshape(NP, 1), dB0.reshape(NP, 1),
                                None, b0.reshape(1, D), W1, al1, ar1, None)
  ee1, dA1, dB1 = _scp1(el1.reshape(NP), er1.reshape(NP), mv1[0, :16],
                        srcp, dstp)
  rst1 = _scp2_128(srcp, dstp, ee1, f1t)
  h2, f2t, res2, el2, er2, mv2 = _tc2(rst1, dA1.reshape(NP, 1),
                                      dB1.reshape(NP, 1), h1,
                                      b1.reshape(1, D), W2, al2, ar2, resW2)
  ee2, dA2, dB2 = _scp1(el2.reshape(NP), er2.reshape(NP), mv2[0, :16],
                        srcp, dstp)
  rst2 = _scp2_128(srcp, dstp, ee2, f2t)[:, :, :NCLS]
  logits = _tc_final(rst2, dA2.reshape(NP, 1), dB2.reshape(NP, 1),
                     res2, b2.reshape(1, NCLS))
  return (logits[:N], h2[:N])
